# speedup vs baseline: 1.1354x; 1.1354x over previous
import functools

import jax
import jax.numpy as jnp
from jax import lax
from jax.experimental import pallas as pl
from jax.experimental.pallas import tpu as pltpu
from jax.experimental.pallas import tpu_sc as plsc

D_MODEL = 2048
MAXLEN = 4096
B = 4 * 4096
NC, NS, L = 2, 16, 16
NW = NC * NS
B_PER_W = B // NW
C = 16
NCHUNK = B_PER_W // C


def _gather_body(idx_hbm, pek_hbm, pev_hbm, ok_hbm, ov_hbm,
                 idx_v, buf0, buf1, sem0, sem1):
    wid = lax.axis_index("s") * NC + lax.axis_index("c")
    base = wid * B_PER_W
    pltpu.sync_copy(idx_hbm.at[pl.ds(base, B_PER_W)], idx_v)

    @pl.loop(0, B_PER_W // L)
    def _(i):
        s = pl.ds(i * L, L)
        v = idx_v[s]
        idx_v[s] = jnp.minimum(jnp.maximum(v, -MAXLEN), MAXLEN - 1) + MAXLEN

    def do_table(tbl_hbm, out_hbm):
        @pl.loop(0, NCHUNK, step=2)
        def _(j):
            h0 = pltpu.async_copy(
                tbl_hbm.at[idx_v.at[pl.ds(j * C, C)]], buf0, sem0)
            h1 = pltpu.async_copy(
                tbl_hbm.at[idx_v.at[pl.ds((j + 1) * C, C)]], buf1, sem1)
            h0.wait()
            pltpu.sync_copy(buf0, out_hbm.at[pl.ds(base + j * C, C)])
            h1.wait()
            pltpu.sync_copy(buf1, out_hbm.at[pl.ds(base + (j + 1) * C, C)])

    do_table(pek_hbm, ok_hbm)
    do_table(pev_hbm, ov_hbm)


@jax.jit
def _run(idx_flat, pe_k, pe_v):
    mesh = plsc.VectorSubcoreMesh(core_axis_name="c", subcore_axis_name="s")
    f = pl.kernel(
        _gather_body,
        mesh=mesh,
        out_type=(jax.ShapeDtypeStruct((B, D_MODEL), jnp.float32),
                  jax.ShapeDtypeStruct((B, D_MODEL), jnp.float32)),
        scratch_types=[
            pltpu.VMEM((B_PER_W,), jnp.int32),
            pltpu.VMEM((C, D_MODEL), jnp.float32),
            pltpu.VMEM((C, D_MODEL), jnp.float32),
            pltpu.SemaphoreType.DMA,
            pltpu.SemaphoreType.DMA,
        ],
    )
    return f(idx_flat, pe_k, pe_v)


def kernel(pos_seq, pe_k, pe_v):
    lead = pos_seq.shape
    idx_flat = pos_seq.reshape(B)
    ok, ov = _run(idx_flat, pe_k, pe_v)
    return (ok.reshape(*lead, D_MODEL), ov.reshape(*lead, D_MODEL))


# speedup vs baseline: 1.1389x; 1.0031x over previous
import functools

import jax
import jax.numpy as jnp
from jax import lax
from jax.experimental import pallas as pl
from jax.experimental.pallas import tpu as pltpu
from jax.experimental.pallas import tpu_sc as plsc

D_MODEL = 2048
MAXLEN = 4096
B = 4 * 4096
NC, NS, L = 2, 16, 16
NW = NC * NS
B_PER_W = B // NW
C = 16
NCHUNK = B_PER_W // C


def _gather_body(idx_hbm, pek_hbm, pev_hbm, ok_hbm, ov_hbm,
                 idx_v, buf0, buf1, gs0, gs1, ws0, ws1):
    wid = lax.axis_index("s") * NC + lax.axis_index("c")
    base = wid * B_PER_W
    pltpu.sync_copy(idx_hbm.at[pl.ds(base, B_PER_W)], idx_v)

    @pl.loop(0, B_PER_W // L)
    def _(i):
        s = pl.ds(i * L, L)
        v = idx_v[s]
        idx_v[s] = jnp.minimum(jnp.maximum(v, -MAXLEN), MAXLEN - 1) + MAXLEN

    bufs = (buf0, buf1)
    gsems = (gs0, gs1)
    wsems = (ws0, ws1)

    def gather(tbl, j, b):
        pltpu.async_copy(tbl.at[idx_v.at[pl.ds(j * C, C)]], bufs[b], gsems[b])

    def do_table(tbl, out):
        gather(tbl, 0, 0)
        gather(tbl, 1, 1)

        @pl.loop(0, NCHUNK, step=2)
        def _(j):
            for b in range(2):
                jj = j + b
                pltpu.make_async_copy(
                    tbl.at[idx_v.at[pl.ds(jj * C, C)]], bufs[b],
                    gsems[b]).wait()
                pltpu.async_copy(
                    bufs[b], out.at[pl.ds(base + jj * C, C)], wsems[b])

                @pl.when(jj + 2 < NCHUNK)
                def _():
                    pltpu.make_async_copy(
                        bufs[b], out.at[pl.ds(base + jj * C, C)],
                        wsems[b]).wait()
                    gather(tbl, jj + 2, b)

        for b in range(2):
            jj = NCHUNK - 2 + b
            pltpu.make_async_copy(
                bufs[b], out.at[pl.ds(base + jj * C, C)], wsems[b]).wait()

    do_table(pek_hbm, ok_hbm)
    do_table(pev_hbm, ov_hbm)


@jax.jit
def _run(idx_flat, pe_k, pe_v):
    mesh = plsc.VectorSubcoreMesh(core_axis_name="c", subcore_axis_name="s")
    f = pl.kernel(
        _gather_body,
        mesh=mesh,
        out_type=(jax.ShapeDtypeStruct((B, D_MODEL), jnp.float32),
                  jax.ShapeDtypeStruct((B, D_MODEL), jnp.float32)),
        scratch_types=[
            pltpu.VMEM((B_PER_W,), jnp.int32),
            pltpu.VMEM((C, D_MODEL), jnp.float32),
            pltpu.VMEM((C, D_MODEL), jnp.float32),
            pltpu.SemaphoreType.DMA,
            pltpu.SemaphoreType.DMA,
            pltpu.SemaphoreType.DMA,
            pltpu.SemaphoreType.DMA,
        ],
    )
    return f(idx_flat, pe_k, pe_v)


def kernel(pos_seq, pe_k, pe_v):
    lead = pos_seq.shape
    idx_flat = pos_seq.reshape(B)
    ok, ov = _run(idx_flat, pe_k, pe_v)
    return (ok.reshape(*lead, D_MODEL), ov.reshape(*lead, D_MODEL))


# speedup vs baseline: 1.1465x; 1.0067x over previous
import functools

import jax
import jax.numpy as jnp
from jax import lax
from jax.experimental import pallas as pl
from jax.experimental.pallas import tpu as pltpu
from jax.experimental.pallas import tpu_sc as plsc

D_MODEL = 2048
MAXLEN = 4096
B = 4 * 4096
NC, NS, L = 2, 16, 16
NW = NC * NS
B_PER_W = B // NW
C = 32
NCHUNK = B_PER_W // C


def _gather_body(idx_hbm, pek_hbm, pev_hbm, ok_hbm, ov_hbm,
                 idx_v, buf0, gs0, ws0):
    wid = lax.axis_index("s") * NC + lax.axis_index("c")
    base = wid * B_PER_W
    pltpu.sync_copy(idx_hbm.at[pl.ds(base, B_PER_W)], idx_v)

    @pl.loop(0, B_PER_W // L)
    def _(i):
        s = pl.ds(i * L, L)
        v = idx_v[s]
        idx_v[s] = jnp.minimum(jnp.maximum(v, -MAXLEN), MAXLEN - 1) + MAXLEN

    def do_table(tbl, out, buf, gsem, wsem):
        @pl.loop(0, NCHUNK)
        def _(j):
            pltpu.async_copy(
                tbl.at[idx_v.at[pl.ds(j * C, C)]], buf, gsem).wait()
            pltpu.sync_copy(buf, out.at[pl.ds(base + j * C, C)])

    do_table(pek_hbm, ok_hbm, buf0, gs0, ws0)
    do_table(pev_hbm, ov_hbm, buf0, gs0, ws0)


@jax.jit
def _run(idx_flat, pe_k, pe_v):
    mesh = plsc.VectorSubcoreMesh(core_axis_name="c", subcore_axis_name="s")
    f = pl.kernel(
        _gather_body,
        mesh=mesh,
        out_type=(jax.ShapeDtypeStruct((B, D_MODEL), jnp.float32),
                  jax.ShapeDtypeStruct((B, D_MODEL), jnp.float32)),
        scratch_types=[
            pltpu.VMEM((B_PER_W,), jnp.int32),
            pltpu.VMEM((C, D_MODEL), jnp.float32),
            pltpu.SemaphoreType.DMA,
            pltpu.SemaphoreType.DMA,
        ],
    )
    return f(idx_flat, pe_k, pe_v)


def kernel(pos_seq, pe_k, pe_v):
    lead = pos_seq.shape
    idx_flat = pos_seq.reshape(B)
    ok, ov = _run(idx_flat, pe_k, pe_v)
    return (ok.reshape(*lead, D_MODEL), ov.reshape(*lead, D_MODEL))


# speedup vs baseline: 1.1990x; 1.0458x over previous
import functools

import jax
import jax.numpy as jnp
from jax import lax
from jax.experimental import pallas as pl
from jax.experimental.pallas import tpu as pltpu
from jax.experimental.pallas import tpu_sc as plsc

D_MODEL = 2048
MAXLEN = 4096
B = 4 * 4096
NC, NS, L = 2, 16, 16
NW = NC * NS
B_PER_W = B // NW
C = 32
NCHUNK = B_PER_W // C

G = 8


def _clamp(v):
    return jnp.minimum(jnp.maximum(v, -MAXLEN), MAXLEN - 1) + MAXLEN


def _sc_body(idx_hbm, tbl_hbm, out_hbm, idx_v, buf, gsem):
    wid = lax.axis_index("s") * NC + lax.axis_index("c")
    base = wid * B_PER_W
    pltpu.sync_copy(idx_hbm.at[pl.ds(base, B_PER_W)], idx_v)

    @pl.loop(0, B_PER_W // L)
    def _(i):
        s = pl.ds(i * L, L)
        idx_v[s] = _clamp(idx_v[s])

    @pl.loop(0, NCHUNK)
    def _(j):
        pltpu.async_copy(
            tbl_hbm.at[idx_v.at[pl.ds(j * C, C)]], buf, gsem).wait()
        pltpu.sync_copy(buf, out_hbm.at[pl.ds(base + j * C, C)])


def _sc_gather(idx_flat, table):
    mesh = plsc.VectorSubcoreMesh(core_axis_name="c", subcore_axis_name="s")
    f = pl.kernel(
        _sc_body,
        mesh=mesh,
        out_type=jax.ShapeDtypeStruct((B, D_MODEL), jnp.float32),
        scratch_types=[
            pltpu.VMEM((B_PER_W,), jnp.int32),
            pltpu.VMEM((C, D_MODEL), jnp.float32),
            pltpu.SemaphoreType.DMA,
        ],
    )
    return f(idx_flat, table)


def _tc_body(idx_ref, *refs):
    out = refs[G]
    for t in range(G):
        out[t, :] = refs[t][0, 0, :]


def _tc_gather(idx_flat, table):
    table3 = table.reshape(table.shape[0], 1, D_MODEL)
    in_specs = [
        pl.BlockSpec(
            (1, 1, D_MODEL),
            (lambda i, idx_ref, t=t: (_clamp(idx_ref[G * i + t]), 0, 0)))
        for t in range(G)
    ]
    out_spec = pl.BlockSpec((G, D_MODEL), lambda i, idx_ref: (i, 0))
    return pl.pallas_call(
        _tc_body,
        grid_spec=pltpu.PrefetchScalarGridSpec(
            num_scalar_prefetch=1,
            grid=(B // G,),
            in_specs=in_specs,
            out_specs=out_spec,
        ),
        out_shape=jax.ShapeDtypeStruct((B, D_MODEL), jnp.float32),
    )(idx_flat, *([table3] * G))


@jax.jit
def _run(idx_flat, pe_k, pe_v):
    return _sc_gather(idx_flat, pe_k), _tc_gather(idx_flat, pe_v)


def kernel(pos_seq, pe_k, pe_v):
    lead = pos_seq.shape
    idx_flat = pos_seq.reshape(B)
    ok, ov = _run(idx_flat, pe_k, pe_v)
    return (ok.reshape(*lead, D_MODEL), ov.reshape(*lead, D_MODEL))


# speedup vs baseline: 1.6249x; 1.3553x over previous
import functools

import jax
import jax.numpy as jnp
from jax import lax
from jax.experimental import pallas as pl
from jax.experimental.pallas import tpu as pltpu
from jax.experimental.pallas import tpu_sc as plsc

D_MODEL = 2048
MAXLEN = 4096
B = 4 * 4096
NC, NS, L = 2, 16, 16
NW = NC * NS
B_PER_W = B // NW
C = 32
NCHUNK = B_PER_W // C

G = 16


def _clamp(v):
    return jnp.minimum(jnp.maximum(v, -MAXLEN), MAXLEN - 1) + MAXLEN


def _sc_body(idx_hbm, tbl_hbm, out_hbm, idx_v, buf, gsem):
    wid = lax.axis_index("s") * NC + lax.axis_index("c")
    base = wid * B_PER_W
    pltpu.sync_copy(idx_hbm.at[pl.ds(base, B_PER_W)], idx_v)

    @pl.loop(0, B_PER_W // L)
    def _(i):
        s = pl.ds(i * L, L)
        idx_v[s] = _clamp(idx_v[s])

    @pl.loop(0, NCHUNK)
    def _(j):
        pltpu.async_copy(
            tbl_hbm.at[idx_v.at[pl.ds(j * C, C)]], buf, gsem).wait()
        pltpu.sync_copy(buf, out_hbm.at[pl.ds(base + j * C, C)])


def _sc_gather(idx_flat, table):
    mesh = plsc.VectorSubcoreMesh(core_axis_name="c", subcore_axis_name="s")
    f = pl.kernel(
        _sc_body,
        mesh=mesh,
        out_type=jax.ShapeDtypeStruct((B, D_MODEL), jnp.float32),
        scratch_types=[
            pltpu.VMEM((B_PER_W,), jnp.int32),
            pltpu.VMEM((C, D_MODEL), jnp.float32),
            pltpu.SemaphoreType.DMA,
        ],
    )
    return f(idx_flat, table)


def _tc_body(idx_ref, *refs):
    out = refs[G]
    for t in range(G):
        out[t, :] = refs[t][0, 0, :]


def _tc_gather(idx_flat, table):
    table3 = table.reshape(table.shape[0], 1, D_MODEL)
    in_specs = [
        pl.BlockSpec(
            (1, 1, D_MODEL),
            (lambda i, idx_ref, t=t: (_clamp(idx_ref[G * i + t]), 0, 0)))
        for t in range(G)
    ]
    out_spec = pl.BlockSpec((G, D_MODEL), lambda i, idx_ref: (i, 0))
    return pl.pallas_call(
        _tc_body,
        grid_spec=pltpu.PrefetchScalarGridSpec(
            num_scalar_prefetch=1,
            grid=(B // G,),
            in_specs=in_specs,
            out_specs=out_spec,
        ),
        out_shape=jax.ShapeDtypeStruct((B, D_MODEL), jnp.float32),
        compiler_params=pltpu.CompilerParams(
            dimension_semantics=("parallel",)),
    )(idx_flat, *([table3] * G))


@jax.jit
def _run(idx_flat, pe_k, pe_v):
    return _sc_gather(idx_flat, pe_k), _tc_gather(idx_flat, pe_v)


def kernel(pos_seq, pe_k, pe_v):
    lead = pos_seq.shape
    idx_flat = pos_seq.reshape(B)
    ok, ov = _run(idx_flat, pe_k, pe_v)
    return (ok.reshape(*lead, D_MODEL), ov.reshape(*lead, D_MODEL))
